# baseline (speedup 1.0000x reference)
import functools

import jax
import jax.numpy as jnp
from jax import lax
from jax.experimental import pallas as pl
from jax.experimental.pallas import tpu as pltpu
from jax.experimental.pallas import tpu_sc as plsc

D_MODEL = 64
SCALE = 8.0

_NC = 2
_NS = 16
_NW = _NC * _NS

_B = 4096 * 200
_BPW = _B // _NW
_C = 128
_NCHUNK = _BPW // _C


@functools.partial(jax.jit, static_argnames=())
def _sc_lookup(x_grouped, lut):
    mesh = plsc.VectorSubcoreMesh(core_axis_name="c", subcore_axis_name="s")

    @functools.partial(
        pl.kernel,
        out_type=jax.ShapeDtypeStruct((_B, D_MODEL), jnp.float32),
        mesh=mesh,
        compiler_params=pltpu.CompilerParams(use_tc_tiling_on_sc=False),
        scratch_types=[
            pltpu.VMEM((_C,), jnp.int32),
            pltpu.VMEM((_C, D_MODEL), jnp.float32),
            pltpu.VMEM((_C, D_MODEL), jnp.float32),
            pltpu.SemaphoreType.DMA,
            pltpu.SemaphoreType.DMA,
        ],
    )
    def k(x_hbm, lut_hbm, out_hbm, idx_v, rows_v, out_v, gsem, ssem):
        wid = lax.axis_index("s") * _NC + lax.axis_index("c")
        base = wid * _BPW

        def chunk(c, carry):
            pltpu.sync_copy(x_hbm.at[wid, c], idx_v)
            pltpu.async_copy(lut_hbm.at[idx_v], rows_v, gsem).wait()

            def row(i, carry2):
                for j in range(4):
                    sl = pl.ds(16 * j, 16)
                    out_v[i, sl] = rows_v[i, sl] * SCALE
                return carry2

            lax.fori_loop(0, _C, row, 0)
            pltpu.async_copy(
                out_v, out_hbm.at[pl.ds(base + c * _C, _C)], ssem
            ).wait()
            return carry

        lax.fori_loop(0, _NCHUNK, chunk, 0)

    return k(x_grouped, lut)


def kernel(x, lut):
    x_grouped = x.reshape(_NW, _NCHUNK, _C).astype(jnp.int32)
    out = _sc_lookup(x_grouped, lut)
    return out.reshape(x.shape[0], x.shape[1], D_MODEL)


# speedup vs baseline: 1.2268x; 1.2268x over previous
import functools

import jax
import jax.numpy as jnp
from jax import lax
from jax.experimental import pallas as pl
from jax.experimental.pallas import tpu as pltpu
from jax.experimental.pallas import tpu_sc as plsc

D_MODEL = 64
SCALE = 8.0

_NC = 2
_NS = 16
_NW = _NC * _NS

_B = 4096 * 200
_BPW = _B // _NW
_C = 128
_NCHUNK = _BPW // _C


def _sc_lookup(x_grouped, lut):
    mesh = plsc.VectorSubcoreMesh(core_axis_name="c", subcore_axis_name="s")

    @functools.partial(
        pl.kernel,
        out_type=jax.ShapeDtypeStruct((_B, D_MODEL), jnp.float32),
        mesh=mesh,
        compiler_params=pltpu.CompilerParams(use_tc_tiling_on_sc=False),
        scratch_types=[
            pltpu.VMEM((_NCHUNK, _C), jnp.int32),
            pltpu.VMEM((_C, D_MODEL), jnp.float32),
            pltpu.VMEM((_C, D_MODEL), jnp.float32),
            pltpu.VMEM((_C, D_MODEL), jnp.float32),
            pltpu.VMEM((_C, D_MODEL), jnp.float32),
            pltpu.SemaphoreType.DMA,
            pltpu.SemaphoreType.DMA,
            pltpu.SemaphoreType.DMA,
            pltpu.SemaphoreType.DMA,
        ],
    )
    def k(x_hbm, lut_hbm, out_hbm, idx_v, a0, a1, b0, b1, g0, g1, s0, s1):
        wid = lax.axis_index("s") * _NC + lax.axis_index("c")
        base = wid * _BPW
        pltpu.sync_copy(x_hbm.at[wid], idx_v)

        abufs, bbufs = (a0, a1), (b0, b1)
        gsems, ssems = (g0, g1), (s0, s1)

        pltpu.async_copy(lut_hbm.at[idx_v.at[0]], a0, g0)
        pltpu.async_copy(lut_hbm.at[idx_v.at[1]], a1, g1)

        def pair(g, carry):
            for b in range(2):
                c = 2 * g + b
                A, Bb, gs, ss = abufs[b], bbufs[b], gsems[b], ssems[b]

                pltpu.make_async_copy(lut_hbm.at[idx_v.at[c]], A, gs).wait()

                @pl.when(g >= 1)
                def _():
                    pltpu.make_async_copy(
                        Bb, out_hbm.at[pl.ds(base + (c - 2) * _C, _C)], ss
                    ).wait()

                def rows4(i, carry2):
                    r = i * 4
                    for u in range(4):
                        for j in range(4):
                            sl = pl.ds(16 * j, 16)
                            Bb[r + u, sl] = A[r + u, sl] * SCALE
                    return carry2

                lax.fori_loop(0, _C // 4, rows4, 0)

                @pl.when(c + 2 < _NCHUNK)
                def _():
                    pltpu.async_copy(lut_hbm.at[idx_v.at[c + 2]], A, gs)

                pltpu.async_copy(
                    Bb, out_hbm.at[pl.ds(base + c * _C, _C)], ss
                )
            return carry

        lax.fori_loop(0, _NCHUNK // 2, pair, 0)

        for b in range(2):
            c = _NCHUNK - 2 + b
            pltpu.make_async_copy(
                bbufs[b], out_hbm.at[pl.ds(base + c * _C, _C)], ssems[b]
            ).wait()

    return k(x_grouped, lut)


def kernel(x, lut):
    x_grouped = x.reshape(_NW, _NCHUNK, _C).astype(jnp.int32)
    out = _sc_lookup(x_grouped, lut)
    return out.reshape(x.shape[0], x.shape[1], D_MODEL)


# speedup vs baseline: 1.3034x; 1.0624x over previous
import functools

import jax
import jax.numpy as jnp
from jax import lax
from jax.experimental import pallas as pl
from jax.experimental.pallas import tpu as pltpu
from jax.experimental.pallas import tpu_sc as plsc

D_MODEL = 64
SCALE = 8.0

_NC = 2
_NS = 16
_NW = _NC * _NS

_V = 1000000
_CA = 128
_NCHA = _V // _CA
_TAILR = _V - _NCHA * _CA
_TAILW = 4

_B = 4096 * 200
_BPW = _B // _NW
_C = 128
_NCHUNK = _BPW // _C

_MESH = dict(core_axis_name="c", subcore_axis_name="s")
_PARAMS = pltpu.CompilerParams(use_tc_tiling_on_sc=True)


def _worker_id():
    return lax.axis_index("s") * _NC + lax.axis_index("c")


def _copy_rows(src, dst, nrows, scale):
    def rows4(i, carry):
        r = i * 4
        for u in range(4):
            for j in range(4):
                sl = pl.ds(16 * j, 16)
                dst[r + u, sl] = src[r + u, sl] * scale
        return carry

    lax.fori_loop(0, nrows // 4, rows4, 0)


@functools.partial(
    pl.kernel,
    out_type=jax.ShapeDtypeStruct((_V, 128), jnp.float32),
    mesh=plsc.VectorSubcoreMesh(**_MESH),
    compiler_params=_PARAMS,
    scratch_types=(
        [pltpu.VMEM((_CA, D_MODEL), jnp.float32) for _ in range(2)]
        + [pltpu.VMEM((_CA, 128), jnp.float32) for _ in range(2)]
        + [pltpu.SemaphoreType.DMA for _ in range(4)]
    ),
)
def _relayout(lut_hbm, t2_hbm, bi0, bi1, bo0, bo1, si0, si1, so0, so1):
    wid = _worker_id()
    bibufs, bobufs = (bi0, bi1), (bo0, bo1)
    isems, osems = (si0, si1), (so0, so1)

    def in_copy(cid, b):
        return pltpu.make_async_copy(
            lut_hbm.at[pl.ds(cid * _CA, _CA)], bibufs[b], isems[b]
        )

    def out_copy(cid, b):
        return pltpu.make_async_copy(
            bobufs[b], t2_hbm.at[pl.ds(cid * _CA, _CA)], osems[b]
        )

    n_k = (_NCHA - 1 - wid) // _NW + 1

    def cid_of(k):
        return wid + _NW * k

    in_copy(cid_of(0), 0).start()
    in_copy(cid_of(1), 1).start()

    def step(k, b):
        in_copy(cid_of(k), b).wait()

        @pl.when(k >= 2)
        def _():
            out_copy(0, b).wait()

        _copy_rows(bibufs[b], bobufs[b], _CA, 1.0)

        @pl.when(k + 2 < n_k)
        def _():
            in_copy(cid_of(k + 2), b).start()

        out_copy(cid_of(k), b).start()

    def pair(g, carry):
        step(2 * g, 0)
        step(2 * g + 1, 1)
        return carry

    lax.fori_loop(0, n_k // 2, pair, 0)

    @pl.when(n_k % 2 == 1)
    def _():
        step(n_k - 1, 0)

    for b in range(2):
        out_copy(0, b).wait()

    @pl.when(wid == _TAILW)
    def _():
        r0 = _NCHA * _CA
        pltpu.make_async_copy(
            lut_hbm.at[pl.ds(r0, _TAILR)], bi0.at[pl.ds(0, _TAILR)], si0
        ).start()
        pltpu.make_async_copy(
            lut_hbm.at[pl.ds(r0, _TAILR)], bi0.at[pl.ds(0, _TAILR)], si0
        ).wait()
        _copy_rows(bi0, bo0, _TAILR, 1.0)
        pltpu.make_async_copy(
            bo0.at[pl.ds(0, _TAILR)], t2_hbm.at[pl.ds(r0, _TAILR)], so0
        ).start()
        pltpu.make_async_copy(
            bo0.at[pl.ds(0, _TAILR)], t2_hbm.at[pl.ds(r0, _TAILR)], so0
        ).wait()


@functools.partial(
    pl.kernel,
    out_type=jax.ShapeDtypeStruct((_B, D_MODEL), jnp.float32),
    mesh=plsc.VectorSubcoreMesh(**_MESH),
    compiler_params=_PARAMS,
    scratch_types=(
        [pltpu.VMEM((_C,), jnp.int32) for _ in range(4)]
        + [pltpu.VMEM((_C, 128), jnp.float32) for _ in range(2)]
        + [pltpu.VMEM((_C, D_MODEL), jnp.float32) for _ in range(2)]
        + [pltpu.SemaphoreType.DMA for _ in range(8)]
    ),
)
def _gather(x_hbm, t2_hbm, out_hbm,
            i0, i1, i2, i3, a0, a1, s0, s1,
            ig0, ig1, ig2, ig3, g0, g1, st0, st1):
    wid = _worker_id()
    base = wid * _BPW
    ibufs, isems = (i0, i1, i2, i3), (ig0, ig1, ig2, ig3)
    abufs, sbufs = (a0, a1), (s0, s1)
    gsems, ssems = (g0, g1), (st0, st1)

    def idx_copy(cn, k):
        return pltpu.make_async_copy(x_hbm.at[wid, cn], ibufs[k], isems[k])

    def gather_copy(k, ab):
        return pltpu.make_async_copy(t2_hbm.at[ibufs[k]], abufs[ab], gsems[ab])

    def store_copy(cn, ab):
        return pltpu.make_async_copy(
            sbufs[ab], out_hbm.at[pl.ds(base + cn * _C, _C)], ssems[ab]
        )

    for k in range(4):
        idx_copy(k, k).start()
    for b in range(2):
        idx_copy(b, b).wait()
        gather_copy(b, b).start()

    def quad(g, carry):
        for b in range(4):
            c = 4 * g + b
            ab = b % 2
            gather_copy(b, ab).wait()

            @pl.when(c + 4 < _NCHUNK)
            def _(c=c, b=b):
                idx_copy(c + 4, b).start()

            @pl.when(c >= 2)
            def _(c=c, ab=ab):
                store_copy(c - 2, ab).wait()

            _copy_rows(abufs[ab], sbufs[ab], _C, SCALE)

            store_copy(c, ab).start()

            @pl.when(c + 2 < _NCHUNK)
            def _(c=c, b=b, ab=ab):
                idx_copy(c + 2, (b + 2) % 4).wait()
                gather_copy((b + 2) % 4, ab).start()
        return carry

    lax.fori_loop(0, _NCHUNK // 4, quad, 0)

    for b in range(2):
        store_copy(_NCHUNK - 2 + b, b).wait()


def kernel(x, lut):
    x_grouped = x.reshape(_NW, _NCHUNK, _C).astype(jnp.int32)
    t2 = _relayout(lut)
    dep = (jnp.sum(t2[0, :1]) * 0.0).astype(jnp.int32)
    out = _gather(jnp.bitwise_or(x_grouped, dep), t2)
    return out.reshape(x.shape[0], x.shape[1], D_MODEL)


# speedup vs baseline: 1.3037x; 1.0003x over previous
import functools

import jax
import jax.numpy as jnp
from jax import lax
from jax.experimental import pallas as pl
from jax.experimental.pallas import tpu as pltpu
from jax.experimental.pallas import tpu_sc as plsc

D_MODEL = 64
SCALE = 8.0

_NC = 2
_NS = 16
_NW = _NC * _NS

_V = 1000000
_CA = 128
_NCHA = _V // _CA
_TAILR = _V - _NCHA * _CA
_TAILW = 4

_B = 4096 * 200
_BPW = _B // _NW
_C = 128
_NCHUNK = _BPW // _C

_MESH = dict(core_axis_name="c", subcore_axis_name="s")
_PARAMS = pltpu.CompilerParams(use_tc_tiling_on_sc=True)


def _worker_id():
    return lax.axis_index("s") * _NC + lax.axis_index("c")


def _copy_rows(src, dst, nrows, scale):
    def rows4(i, carry):
        r = i * 4
        for u in range(4):
            for j in range(4):
                sl = pl.ds(16 * j, 16)
                dst[r + u, sl] = src[r + u, sl] * scale
        return carry

    lax.fori_loop(0, nrows // 4, rows4, 0)


@functools.partial(
    pl.kernel,
    out_type=jax.ShapeDtypeStruct((_V, 128), jnp.float32),
    mesh=plsc.VectorSubcoreMesh(**_MESH),
    compiler_params=_PARAMS,
    scratch_types=(
        [pltpu.VMEM((_CA, D_MODEL), jnp.float32) for _ in range(2)]
        + [pltpu.VMEM((_CA, 128), jnp.float32) for _ in range(2)]
        + [pltpu.SemaphoreType.DMA for _ in range(4)]
    ),
)
def _relayout(lut_hbm, t2_hbm, bi0, bi1, bo0, bo1, si0, si1, so0, so1):
    wid = _worker_id()
    bibufs, bobufs = (bi0, bi1), (bo0, bo1)
    isems, osems = (si0, si1), (so0, so1)

    def in_copy(cid, b):
        return pltpu.make_async_copy(
            lut_hbm.at[pl.ds(cid * _CA, _CA)], bibufs[b], isems[b]
        )

    def out_copy(cid, b):
        return pltpu.make_async_copy(
            bobufs[b], t2_hbm.at[pl.ds(cid * _CA, _CA)], osems[b]
        )

    n_k = (_NCHA - 1 - wid) // _NW + 1

    def cid_of(k):
        return wid + _NW * k

    in_copy(cid_of(0), 0).start()
    in_copy(cid_of(1), 1).start()

    def step(k, b):
        in_copy(cid_of(k), b).wait()

        @pl.when(k >= 2)
        def _():
            out_copy(0, b).wait()

        _copy_rows(bibufs[b], bobufs[b], _CA, SCALE)

        @pl.when(k + 2 < n_k)
        def _():
            in_copy(cid_of(k + 2), b).start()

        out_copy(cid_of(k), b).start()

    def pair(g, carry):
        step(2 * g, 0)
        step(2 * g + 1, 1)
        return carry

    lax.fori_loop(0, n_k // 2, pair, 0)

    @pl.when(n_k % 2 == 1)
    def _():
        step(n_k - 1, 0)

    for b in range(2):
        out_copy(0, b).wait()

    @pl.when(wid == _TAILW)
    def _():
        r0 = _NCHA * _CA
        pltpu.make_async_copy(
            lut_hbm.at[pl.ds(r0, _TAILR)], bi0.at[pl.ds(0, _TAILR)], si0
        ).start()
        pltpu.make_async_copy(
            lut_hbm.at[pl.ds(r0, _TAILR)], bi0.at[pl.ds(0, _TAILR)], si0
        ).wait()
        _copy_rows(bi0, bo0, _TAILR, SCALE)
        pltpu.make_async_copy(
            bo0.at[pl.ds(0, _TAILR)], t2_hbm.at[pl.ds(r0, _TAILR)], so0
        ).start()
        pltpu.make_async_copy(
            bo0.at[pl.ds(0, _TAILR)], t2_hbm.at[pl.ds(r0, _TAILR)], so0
        ).wait()


@functools.partial(
    pl.kernel,
    out_type=jax.ShapeDtypeStruct((_B, D_MODEL), jnp.float32),
    mesh=plsc.VectorSubcoreMesh(**_MESH),
    compiler_params=_PARAMS,
    scratch_types=(
        [pltpu.VMEM((_C,), jnp.int32) for _ in range(4)]
        + [pltpu.VMEM((_C, 128), jnp.float32) for _ in range(2)]
        + [pltpu.VMEM((_C, D_MODEL), jnp.float32) for _ in range(2)]
        + [pltpu.SemaphoreType.DMA for _ in range(8)]
    ),
)
def _gather(x_hbm, t2_hbm, out_hbm,
            i0, i1, i2, i3, a0, a1, s0, s1,
            ig0, ig1, ig2, ig3, g0, g1, st0, st1):
    wid = _worker_id()
    base = wid * _BPW
    ibufs, isems = (i0, i1, i2, i3), (ig0, ig1, ig2, ig3)
    abufs, sbufs = (a0, a1), (s0, s1)
    gsems, ssems = (g0, g1), (st0, st1)

    def idx_copy(cn, k):
        return pltpu.make_async_copy(x_hbm.at[wid, cn], ibufs[k], isems[k])

    def gather_copy(k, ab):
        return pltpu.make_async_copy(t2_hbm.at[ibufs[k]], abufs[ab], gsems[ab])

    def store_copy(cn, ab):
        return pltpu.make_async_copy(
            sbufs[ab], out_hbm.at[pl.ds(base + cn * _C, _C)], ssems[ab]
        )

    for k in range(4):
        idx_copy(k, k).start()
    for b in range(2):
        idx_copy(b, b).wait()
        gather_copy(b, b).start()

    def quad(g, carry):
        for b in range(4):
            c = 4 * g + b
            ab = b % 2
            gather_copy(b, ab).wait()

            @pl.when(c + 4 < _NCHUNK)
            def _(c=c, b=b):
                idx_copy(c + 4, b).start()

            @pl.when(c >= 2)
            def _(c=c, ab=ab):
                store_copy(c - 2, ab).wait()

            _copy_rows(abufs[ab], sbufs[ab], _C, 1.0)

            store_copy(c, ab).start()

            @pl.when(c + 2 < _NCHUNK)
            def _(c=c, b=b, ab=ab):
                idx_copy(c + 2, (b + 2) % 4).wait()
                gather_copy((b + 2) % 4, ab).start()
        return carry

    lax.fori_loop(0, _NCHUNK // 4, quad, 0)

    for b in range(2):
        store_copy(_NCHUNK - 2 + b, b).wait()


def kernel(x, lut):
    x_grouped = x.reshape(_NW, _NCHUNK, _C).astype(jnp.int32)
    t2 = _relayout(lut)
    dep = (jnp.sum(t2[0, :1]) * 0.0).astype(jnp.int32)
    out = _gather(jnp.bitwise_or(x_grouped, dep), t2)
    return out.reshape(x.shape[0], x.shape[1], D_MODEL)
